# baseline (speedup 1.0000x reference)
import functools

import jax
import jax.numpy as jnp
from jax import lax
from jax.experimental import pallas as pl
from jax.experimental.pallas import tpu as pltpu
from jax.experimental.pallas import tpu_sc as plsc

NB = 15
NC = 32
L = 16
NCORE = 2
NSUB = 16
NW = NCORE * NSUB
N_ROWS = 1000000
CHUNK = 1000
NCHUNK = N_ROWS // CHUNK
MAXC = 32
TBL = NB * NC


def _sc_body(probas_hbm, labels_hbm, edges_hbm, out_hbm,
             pb0, pb1, lb0, lb1, ebuf, cnt, sump, acc, sem0, sem1):
    w = lax.axis_index("s") * NCORE + lax.axis_index("c")
    base = 31 * w + jnp.minimum(w, 8)
    n_my = jnp.where(w < 8, 32, 31)

    pltpu.sync_copy(edges_hbm, ebuf)

    zeros = jnp.zeros((L,), jnp.float32)
    for tbl in (cnt, sump, acc):
        for j in range(TBL // L):
            tbl[pl.ds(j * L, L)] = zeros

    iota = lax.iota(jnp.int32, L)
    cvecs = (iota, iota + L)
    ones = jnp.ones((L,), jnp.float32)

    pbufs = (pb0, pb1)
    lbufs = (lb0, lb1)
    sems = (sem0, sem1)

    def start(slot, b):
        row0 = (base + slot) * CHUNK
        dp = pltpu.async_copy(
            probas_hbm.at[pl.ds(row0 * NC, CHUNK * NC)], pbufs[b], sems[b])
        dl = pltpu.async_copy(
            labels_hbm.at[pl.ds(row0, CHUNK)], lbufs[b], sems[b])
        return (dp, dl)

    def compute(pb, lb):
        def row(r, carry):
            lblv = plsc.load_gather(lb, [jnp.full((L,), r, jnp.int32)])
            for h in range(2):
                cvec = cvecs[h]
                p = pb[pl.ds(r * NC + h * L, L)]
                j0 = jnp.minimum((p * float(NB)).astype(jnp.int32), NB - 1)
                lo = plsc.load_gather(ebuf, [j0])
                hi = plsc.load_gather(ebuf, [j0 + 1])
                jf = j0 + (p > hi).astype(jnp.int32) - (p < lo).astype(jnp.int32)
                valid = (p != lo) & (p != hi)
                idx = jf * NC + cvec
                plsc.addupdate_scatter(cnt, [idx], ones, mask=valid)
                plsc.addupdate_scatter(sump, [idx], p, mask=valid)
                plsc.addupdate_scatter(acc, [idx], ones,
                                       mask=valid & (lblv == cvec))
            return carry
        lax.fori_loop(0, CHUNK, row, 0)

    pending = [start(0, 0), start(1, 1)]
    for i in range(MAXC):
        b = i % 2
        if i < MAXC - 1:
            for d in pending[b]:
                d.wait()
            compute(pbufs[b], lbufs[b])
            if i + 2 < MAXC - 1:
                pending[b] = start(i + 2, b)
            elif i + 2 == MAXC - 1:
                @pl.when(n_my > MAXC - 1)
                def _():
                    for d in start(i + 2, b):
                        d.wait()
                    compute(pbufs[b], lbufs[b])

    for t, tbl in ((0, cnt), (1, sump), (2, acc)):
        pltpu.sync_copy(tbl, out_hbm.at[pl.ds((w * 3 + t) * TBL, TBL)])


_sc_hist = functools.partial(
    pl.kernel,
    out_type=jax.ShapeDtypeStruct((NW * 3 * TBL,), jnp.float32),
    mesh=plsc.VectorSubcoreMesh(core_axis_name="c", subcore_axis_name="s",
                                num_cores=NCORE, num_subcores=NSUB),
    compiler_params=pltpu.CompilerParams(needs_layout_passes=False),
    scratch_types=[
        pltpu.VMEM((CHUNK * NC,), jnp.float32),
        pltpu.VMEM((CHUNK * NC,), jnp.float32),
        pltpu.VMEM((CHUNK,), jnp.int32),
        pltpu.VMEM((CHUNK,), jnp.int32),
        pltpu.VMEM((L,), jnp.float32),
        pltpu.VMEM((TBL,), jnp.float32),
        pltpu.VMEM((TBL,), jnp.float32),
        pltpu.VMEM((TBL,), jnp.float32),
        pltpu.SemaphoreType.DMA,
        pltpu.SemaphoreType.DMA,
    ],
)(_sc_body)


def _tc_final_body(cp_ref, sp_ref, ap_ref, out_ref):
    cnt = jnp.sum(cp_ref[...], axis=0)
    sp = jnp.sum(sp_ref[...], axis=0)
    ac = jnp.sum(ap_ref[...], axis=0)
    tot = jnp.sum(cnt, axis=0, keepdims=True)
    dc = sp - ac
    pos = cnt > 0
    den = jnp.where(pos, cnt * tot, 1.0)
    term = jnp.where(pos, (dc * dc) / den, 0.0)
    out_ref[0, 0] = jnp.sqrt(jnp.sum(term) / float(NC))


_tc_final = pl.pallas_call(
    _tc_final_body,
    out_shape=jax.ShapeDtypeStruct((1, 1), jnp.float32),
    out_specs=pl.BlockSpec(memory_space=pltpu.SMEM),
)


def kernel(probas, labels):
    edges = jnp.linspace(0.0, 1.0, NB + 1, dtype=jnp.float32)
    parts = _sc_hist(probas.reshape(N_ROWS * NC), labels, edges)
    parts = parts.reshape(NW, 3, NB, NC)
    mce = _tc_final(parts[:, 0], parts[:, 1], parts[:, 2])
    return mce[0, 0]


# speedup vs baseline: 1.7168x; 1.7168x over previous
import functools

import jax
import jax.numpy as jnp
from jax import lax
from jax.experimental import pallas as pl
from jax.experimental.pallas import tpu as pltpu
from jax.experimental.pallas import tpu_sc as plsc

NB = 15
NC = 32
L = 16
NCORE = 2
NSUB = 16
NW = NCORE * NSUB
N_ROWS = 1000000
CHUNK = 1000
NCHUNK = N_ROWS // CHUNK
MAXC = 32
TBL = NB * NC


def _sc_body(probas_hbm, labels_hbm, edges_hbm, out_hbm,
             pb0, pb1, lb0, lb1, ebuf, cnt, sump, acc, sem0, sem1):
    w = lax.axis_index("s") * NCORE + lax.axis_index("c")
    base = 31 * w + jnp.minimum(w, 8)
    n_my = jnp.where(w < 8, 32, 31)

    pltpu.sync_copy(edges_hbm, ebuf)

    zeros = jnp.zeros((L,), jnp.float32)
    for tbl in (cnt, sump, acc):
        for j in range(TBL // L):
            tbl[pl.ds(j * L, L)] = zeros

    iota = lax.iota(jnp.int32, L)
    cvecs = (iota, iota + L)
    ones = jnp.ones((L,), jnp.float32)

    pbufs = (pb0, pb1)
    lbufs = (lb0, lb1)
    sems = (sem0, sem1)

    def start(slot, b):
        row0 = (base + slot) * CHUNK
        dp = pltpu.async_copy(
            probas_hbm.at[pl.ds(row0 * NC, CHUNK * NC)], pbufs[b], sems[b])
        dl = pltpu.async_copy(
            labels_hbm.at[pl.ds(row0, CHUNK)], lbufs[b], sems[b])
        return (dp, dl)

    def compute(pb, lb):
        @plsc.parallel_loop(0, CHUNK, unroll=8)
        def _(r):
            lblv = plsc.load_gather(lb, [jnp.full((L,), r, jnp.int32)])
            for h in range(2):
                cvec = cvecs[h]
                p = pb[pl.ds(r * NC + h * L, L)]
                j0 = jnp.minimum((p * float(NB)).astype(jnp.int32), NB - 1)
                lo = plsc.load_gather(ebuf, [j0])
                hi = plsc.load_gather(ebuf, [j0 + 1])
                jf = (j0 + (p > hi).astype(jnp.int32)
                      - (p < lo).astype(jnp.int32))
                valid = (p != lo) & (p != hi)
                idx = jf * NC + cvec
                plsc.addupdate_scatter(cnt, [idx], ones, mask=valid)
                plsc.addupdate_scatter(sump, [idx], p, mask=valid)
                plsc.addupdate_scatter(acc, [idx], ones,
                                       mask=valid & (lblv == cvec))

    def drain(b):
        pltpu.make_async_copy(
            probas_hbm.at[pl.ds(0, CHUNK * NC)], pbufs[b], sems[b]).wait()
        pltpu.make_async_copy(
            labels_hbm.at[pl.ds(0, CHUNK)], lbufs[b], sems[b]).wait()

    start(0, 0)
    start(1, 1)

    def ring(i, carry):
        for b in range(2):
            chunk = 2 * i + b

            @pl.when(chunk < n_my)
            def _():
                drain(b)
                compute(pbufs[b], lbufs[b])

                @pl.when(chunk + 2 < n_my)
                def _():
                    start(chunk + 2, b)
        return carry

    lax.fori_loop(0, MAXC // 2, ring, 0)

    for t, tbl in ((0, cnt), (1, sump), (2, acc)):
        pltpu.sync_copy(tbl, out_hbm.at[pl.ds((w * 3 + t) * TBL, TBL)])


_sc_hist = functools.partial(
    pl.kernel,
    out_type=jax.ShapeDtypeStruct((NW * 3 * TBL,), jnp.float32),
    mesh=plsc.VectorSubcoreMesh(core_axis_name="c", subcore_axis_name="s",
                                num_cores=NCORE, num_subcores=NSUB),
    compiler_params=pltpu.CompilerParams(needs_layout_passes=False),
    scratch_types=[
        pltpu.VMEM((CHUNK * NC,), jnp.float32),
        pltpu.VMEM((CHUNK * NC,), jnp.float32),
        pltpu.VMEM((CHUNK,), jnp.int32),
        pltpu.VMEM((CHUNK,), jnp.int32),
        pltpu.VMEM((L,), jnp.float32),
        pltpu.VMEM((TBL,), jnp.float32),
        pltpu.VMEM((TBL,), jnp.float32),
        pltpu.VMEM((TBL,), jnp.float32),
        pltpu.SemaphoreType.DMA,
        pltpu.SemaphoreType.DMA,
    ],
)(_sc_body)


def _tc_final_body(cp_ref, sp_ref, ap_ref, out_ref):
    cnt = jnp.sum(cp_ref[...], axis=0)
    sp = jnp.sum(sp_ref[...], axis=0)
    ac = jnp.sum(ap_ref[...], axis=0)
    tot = jnp.sum(cnt, axis=0, keepdims=True)
    dc = sp - ac
    pos = cnt > 0
    den = jnp.where(pos, cnt * tot, 1.0)
    term = jnp.where(pos, (dc * dc) / den, 0.0)
    out_ref[0, 0] = jnp.sqrt(jnp.sum(term) / float(NC))


_tc_final = pl.pallas_call(
    _tc_final_body,
    out_shape=jax.ShapeDtypeStruct((1, 1), jnp.float32),
    out_specs=pl.BlockSpec(memory_space=pltpu.SMEM),
)


def kernel(probas, labels):
    edges = jnp.linspace(0.0, 1.0, NB + 1, dtype=jnp.float32)
    parts = _sc_hist(probas.reshape(N_ROWS * NC), labels, edges)
    parts = parts.reshape(NW, 3, NB, NC)
    mce = _tc_final(parts[:, 0], parts[:, 1], parts[:, 2])
    return mce[0, 0]


# speedup vs baseline: 2.3115x; 1.3463x over previous
import functools

import jax
import jax.numpy as jnp
from jax import lax
from jax.experimental import pallas as pl
from jax.experimental.pallas import tpu as pltpu
from jax.experimental.pallas import tpu_sc as plsc

NB = 15
NC = 32
L = 16
NCORE = 2
NSUB = 16
NW = NCORE * NSUB
N_ROWS = 1000000
CHUNK = 1000
NCHUNK = N_ROWS // CHUNK
MAXC = 32
TBL = NB * NC
EPAD = 32
PACK = 65536


def _sc_body(probas_hbm, labels_hbm, elo_hbm, ehi_hbm,
             outpk_hbm, outsp_hbm,
             pb0, pb1, lb0, lb1, elo, ehi, pk, sump, sem0, sem1):
    w = lax.axis_index("s") * NCORE + lax.axis_index("c")
    base = 31 * w + jnp.minimum(w, 8)
    n_my = jnp.where(w < 8, 32, 31)

    pltpu.sync_copy(elo_hbm, elo)
    pltpu.sync_copy(ehi_hbm, ehi)

    zf = jnp.zeros((L,), jnp.float32)
    zi = jnp.zeros((L,), jnp.int32)
    for j in range(TBL // L):
        pk[pl.ds(j * L, L)] = zi
        sump[pl.ds(j * L, L)] = zf

    iota = lax.iota(jnp.int32, L)
    cvecs = (iota, iota + L)

    pbufs = (pb0, pb1)
    lbufs = (lb0, lb1)
    sems = (sem0, sem1)

    def start(slot, b):
        row0 = (base + slot) * CHUNK
        pltpu.async_copy(
            probas_hbm.at[pl.ds(row0 * NC, CHUNK * NC)], pbufs[b], sems[b])
        pltpu.async_copy(
            labels_hbm.at[pl.ds(row0, CHUNK)], lbufs[b], sems[b])

    def drain(b):
        pltpu.make_async_copy(
            probas_hbm.at[pl.ds(0, CHUNK * NC)], pbufs[b], sems[b]).wait()
        pltpu.make_async_copy(
            labels_hbm.at[pl.ds(0, CHUNK)], lbufs[b], sems[b]).wait()

    def compute(pb, lb):
        @plsc.parallel_loop(0, CHUNK, unroll=8)
        def _(r):
            lblv = plsc.load_gather(lb, [jnp.full((L,), r, jnp.int32)])
            for h in range(2):
                cvec = cvecs[h]
                p = pb[pl.ds(r * NC + h * L, L)]
                j0 = (p * float(NB)).astype(jnp.int32)
                lo = plsc.load_gather(elo, [j0])
                hi = plsc.load_gather(ehi, [j0])
                valid = jnp.logical_not((p == lo) | (p == hi))
                offs = jnp.where(p > hi, jnp.int32(NC),
                                 jnp.where(p < lo, jnp.int32(-NC),
                                           jnp.int32(0)))
                idx = j0 * NC + cvec + offs
                packv = jnp.where(lblv == cvec, jnp.int32(PACK + 1),
                                  jnp.int32(PACK))
                plsc.addupdate_scatter(pk, [idx], packv, mask=valid)
                plsc.addupdate_scatter(sump, [idx], p, mask=valid)

    start(0, 0)
    start(1, 1)

    def ring(i, carry):
        for b in range(2):
            chunk = 2 * i + b

            @pl.when(chunk < n_my)
            def _():
                drain(b)
                compute(pbufs[b], lbufs[b])

                @pl.when(chunk + 2 < n_my)
                def _():
                    start(chunk + 2, b)
        return carry

    lax.fori_loop(0, MAXC // 2, ring, 0)

    pltpu.sync_copy(pk, outpk_hbm.at[pl.ds(w * TBL, TBL)])
    pltpu.sync_copy(sump, outsp_hbm.at[pl.ds(w * TBL, TBL)])


_sc_hist = functools.partial(
    pl.kernel,
    out_type=(jax.ShapeDtypeStruct((NW * TBL,), jnp.int32),
              jax.ShapeDtypeStruct((NW * TBL,), jnp.float32)),
    mesh=plsc.VectorSubcoreMesh(core_axis_name="c", subcore_axis_name="s",
                                num_cores=NCORE, num_subcores=NSUB),
    compiler_params=pltpu.CompilerParams(needs_layout_passes=False),
    scratch_types=[
        pltpu.VMEM((CHUNK * NC,), jnp.float32),
        pltpu.VMEM((CHUNK * NC,), jnp.float32),
        pltpu.VMEM((CHUNK,), jnp.int32),
        pltpu.VMEM((CHUNK,), jnp.int32),
        pltpu.VMEM((EPAD,), jnp.float32),
        pltpu.VMEM((EPAD,), jnp.float32),
        pltpu.VMEM((TBL,), jnp.int32),
        pltpu.VMEM((TBL,), jnp.float32),
        pltpu.SemaphoreType.DMA,
        pltpu.SemaphoreType.DMA,
    ],
)(_sc_body)


def _tc_final_body(pk_ref, sp_ref, out_ref):
    x = pk_ref[...]
    cnt = jnp.sum((x >> 16).astype(jnp.float32), axis=0)
    ac = jnp.sum((x & 0xFFFF).astype(jnp.float32), axis=0)
    sp = jnp.sum(sp_ref[...], axis=0)
    tot = jnp.sum(cnt, axis=0, keepdims=True)
    dc = sp - ac
    pos = cnt > 0
    den = jnp.where(pos, cnt * tot, 1.0)
    term = jnp.where(pos, (dc * dc) / den, 0.0)
    out_ref[0, 0] = jnp.sqrt(jnp.sum(term) / float(NC))


_tc_final = pl.pallas_call(
    _tc_final_body,
    out_shape=jax.ShapeDtypeStruct((1, 1), jnp.float32),
    out_specs=pl.BlockSpec(memory_space=pltpu.SMEM),
)


def kernel(probas, labels):
    edges = jnp.linspace(0.0, 1.0, NB + 1, dtype=jnp.float32)
    pad = jnp.full((EPAD - (NB + 2),), 2.0, jnp.float32)
    elo = jnp.concatenate([edges, jnp.float32(2.0)[None], pad])
    ehi = jnp.concatenate([edges[1:], jnp.full((2,), 2.0, jnp.float32), pad])
    pk, sp = _sc_hist(probas.reshape(N_ROWS * NC), labels, elo, ehi)
    mce = _tc_final(pk.reshape(NW, NB, NC), sp.reshape(NW, NB, NC))
    return mce[0, 0]


# speedup vs baseline: 3.2037x; 1.3860x over previous
import functools

import jax
import jax.numpy as jnp
from jax import lax
from jax.experimental import pallas as pl
from jax.experimental.pallas import tpu as pltpu
from jax.experimental.pallas import tpu_sc as plsc

NB = 15
NC = 32
L = 16
NCORE = 2
NSUB = 16
NW = NCORE * NSUB
N_ROWS = 1000000
CHUNK = 400
NCHUNK = N_ROWS // CHUNK
MAXC = 80
TBL = NB * NC
EPAD = 32
PACK = 65536


def _sc_body(probas_hbm, labels_hbm, elo_hbm, ehi_hbm,
             outpk_hbm, outsp_hbm,
             pb0, pb1, lb0, lb1, elo, ehi, pk, sump, sem0, sem1):
    w = lax.axis_index("s") * NCORE + lax.axis_index("c")
    base = 78 * w + jnp.minimum(w, 4)
    n_my = jnp.where(w < 4, 79, 78)

    pltpu.sync_copy(elo_hbm, elo)
    pltpu.sync_copy(ehi_hbm, ehi)

    zf = jnp.zeros((L,), jnp.float32)
    zi = jnp.zeros((L,), jnp.int32)
    for j in range(TBL // L):
        pk[pl.ds(j * L, L)] = zi
        sump[pl.ds(j * L, L)] = zf

    iota = lax.iota(jnp.int32, L)
    cvecs = (iota, iota + L)

    pbufs = (pb0, pb1)
    lbufs = (lb0, lb1)
    sems = (sem0, sem1)

    def start(slot, b):
        row0 = (base + slot) * CHUNK
        pltpu.async_copy(
            probas_hbm.at[pl.ds(row0, CHUNK)], pbufs[b], sems[b])
        pltpu.async_copy(
            labels_hbm.at[pl.ds(row0, CHUNK)], lbufs[b], sems[b])

    def drain(b):
        pltpu.make_async_copy(
            probas_hbm.at[pl.ds(0, CHUNK)], pbufs[b], sems[b]).wait()
        pltpu.make_async_copy(
            labels_hbm.at[pl.ds(0, CHUNK)], lbufs[b], sems[b]).wait()

    def compute(pb, lb):
        @plsc.parallel_loop(0, CHUNK, unroll=8)
        def _(r):
            lblv = plsc.load_gather(lb, [jnp.full((L,), r, jnp.int32)])
            for h in range(2):
                cvec = cvecs[h]
                p = pb[r, pl.ds(h * L, L)]
                j0 = (p * float(NB)).astype(jnp.int32)
                lo = plsc.load_gather(elo, [j0])
                hi = plsc.load_gather(ehi, [j0])
                valid = jnp.logical_not((p == lo) | (p == hi))
                offs = jnp.where(p > hi, jnp.int32(NC),
                                 jnp.where(p < lo, jnp.int32(-NC),
                                           jnp.int32(0)))
                idx = j0 * NC + cvec + offs
                packv = jnp.where(lblv == cvec, jnp.int32(PACK + 1),
                                  jnp.int32(PACK))
                plsc.addupdate_scatter(pk, [idx], packv, mask=valid)
                plsc.addupdate_scatter(sump, [idx], p, mask=valid)

    start(0, 0)
    start(1, 1)

    def ring(i, carry):
        for b in range(2):
            chunk = 2 * i + b

            @pl.when(chunk < n_my)
            def _():
                drain(b)
                compute(pbufs[b], lbufs[b])

                @pl.when(chunk + 2 < n_my)
                def _():
                    start(chunk + 2, b)
        return carry

    lax.fori_loop(0, MAXC // 2, ring, 0)

    pltpu.sync_copy(pk, outpk_hbm.at[pl.ds(w * TBL, TBL)])
    pltpu.sync_copy(sump, outsp_hbm.at[pl.ds(w * TBL, TBL)])


_sc_hist = functools.partial(
    pl.kernel,
    out_type=(jax.ShapeDtypeStruct((NW * TBL,), jnp.int32),
              jax.ShapeDtypeStruct((NW * TBL,), jnp.float32)),
    mesh=plsc.VectorSubcoreMesh(core_axis_name="c", subcore_axis_name="s",
                                num_cores=NCORE, num_subcores=NSUB),
    compiler_params=pltpu.CompilerParams(needs_layout_passes=False,
                                         use_tc_tiling_on_sc=True),
    scratch_types=[
        pltpu.VMEM((CHUNK, NC), jnp.float32),
        pltpu.VMEM((CHUNK, NC), jnp.float32),
        pltpu.VMEM((CHUNK,), jnp.int32),
        pltpu.VMEM((CHUNK,), jnp.int32),
        pltpu.VMEM((EPAD,), jnp.float32),
        pltpu.VMEM((EPAD,), jnp.float32),
        pltpu.VMEM((TBL,), jnp.int32),
        pltpu.VMEM((TBL,), jnp.float32),
        pltpu.SemaphoreType.DMA,
        pltpu.SemaphoreType.DMA,
    ],
)(_sc_body)


def _tc_final_body(pk_ref, sp_ref, out_ref):
    x = pk_ref[...]
    cnt = jnp.sum((x >> 16).astype(jnp.float32), axis=0)
    ac = jnp.sum((x & 0xFFFF).astype(jnp.float32), axis=0)
    sp = jnp.sum(sp_ref[...], axis=0)
    tot = jnp.sum(cnt, axis=0, keepdims=True)
    dc = sp - ac
    pos = cnt > 0
    den = jnp.where(pos, cnt * tot, 1.0)
    term = jnp.where(pos, (dc * dc) / den, 0.0)
    out_ref[0, 0] = jnp.sqrt(jnp.sum(term) / float(NC))


_tc_final = pl.pallas_call(
    _tc_final_body,
    out_shape=jax.ShapeDtypeStruct((1, 1), jnp.float32),
    out_specs=pl.BlockSpec(memory_space=pltpu.SMEM),
)


def kernel(probas, labels):
    edges = jnp.linspace(0.0, 1.0, NB + 1, dtype=jnp.float32)
    pad = jnp.full((EPAD - (NB + 2),), 2.0, jnp.float32)
    elo = jnp.concatenate([edges, jnp.float32(2.0)[None], pad])
    ehi = jnp.concatenate([edges[1:], jnp.full((2,), 2.0, jnp.float32), pad])
    pk, sp = _sc_hist(probas, labels, elo, ehi)
    mce = _tc_final(pk.reshape(NW, NB, NC), sp.reshape(NW, NB, NC))
    return mce[0, 0]
